# Initial kernel scaffold; baseline (speedup 1.0000x reference)
import jax
import jax.numpy as jnp
from jax.experimental import pallas as pl


def kernel(x, Wr, W1, W2):
    raise NotImplementedError("write your pallas kernel here")


# speedup vs baseline: 1.3197x; 1.3197x over previous
import jax
import jax.numpy as jnp
from jax.experimental import pallas as pl


def _router_body(x_ref, wrt_ref, g_ref):
    z = jnp.dot(x_ref[...], wrt_ref[...], preferred_element_type=jnp.float32)
    num_e = z.shape[1]
    iota = jax.lax.broadcasted_iota(jnp.int32, z.shape, 1)
    m1 = jnp.max(z, axis=1, keepdims=True)
    i1 = jnp.min(jnp.where(z == m1, iota, num_e), axis=1, keepdims=True)
    is1 = iota == i1
    z2 = jnp.where(is1, -jnp.inf, z)
    m2 = jnp.max(z2, axis=1, keepdims=True)
    i2 = jnp.min(jnp.where(z2 == m2, iota, num_e), axis=1, keepdims=True)
    is2 = iota == i2
    t = jnp.exp(m2 - m1)
    g1 = 1.0 / (1.0 + t)
    g2 = t / (1.0 + t)
    g_ref[...] = jnp.where(is1, g1, jnp.where(is2, g2, 0.0))


def _ffn_body(xbf_ref, g_ref, w1t_ref, w2t_ref, o_ref):
    e = pl.program_id(0)
    hidden = w2t_ref.shape[1]
    h = jnp.dot(xbf_ref[...], w1t_ref[0], preferred_element_type=jnp.float32)
    a = h[:, :hidden]
    b = h[:, hidden:]
    act = (a * jax.nn.sigmoid(a) * b).astype(jnp.bfloat16)
    y = jnp.dot(act, w2t_ref[0], preferred_element_type=jnp.float32)
    contrib = g_ref[0] * y

    @pl.when(e == 0)
    def _():
        o_ref[...] = contrib

    @pl.when(e > 0)
    def _():
        o_ref[...] += contrib


def kernel(x, Wr, W1, W2):
    B, T, D = x.shape
    num_e, two_h, _ = W1.shape
    hidden = W2.shape[2]
    x2 = x.reshape(T, D)
    xbf = x2.astype(jnp.bfloat16)
    W1T = jnp.transpose(W1, (0, 2, 1)).astype(jnp.bfloat16)
    W2T = jnp.transpose(W2, (0, 2, 1)).astype(jnp.bfloat16)
    WrT = Wr.T

    gates = pl.pallas_call(
        _router_body,
        grid=(1,),
        in_specs=[
            pl.BlockSpec((T, D), lambda i: (0, 0)),
            pl.BlockSpec((D, num_e), lambda i: (0, 0)),
        ],
        out_specs=pl.BlockSpec((T, num_e), lambda i: (0, 0)),
        out_shape=jax.ShapeDtypeStruct((T, num_e), jnp.float32),
    )(x2, WrT)

    gcol = gates.T.reshape(num_e, T, 1)

    out = pl.pallas_call(
        _ffn_body,
        grid=(num_e,),
        in_specs=[
            pl.BlockSpec((T, D), lambda e: (0, 0)),
            pl.BlockSpec((1, T, 1), lambda e: (e, 0, 0)),
            pl.BlockSpec((1, D, two_h), lambda e: (e, 0, 0)),
            pl.BlockSpec((1, hidden, D), lambda e: (e, 0, 0)),
        ],
        out_specs=pl.BlockSpec((T, D), lambda e: (0, 0)),
        out_shape=jax.ShapeDtypeStruct((T, D), jnp.float32),
    )(xbf, gcol, W1T, W2T)

    return out.reshape(B, T, D)


# speedup vs baseline: 1.5664x; 1.1870x over previous
import jax
import jax.numpy as jnp
from jax.experimental import pallas as pl
from jax.experimental.pallas import tpu as pltpu


def _moe_body(xt32_ref, xtbf_ref, wr_ref, w1_ref, w2_ref, o_ref, gates_s):
    e = pl.program_id(0)
    num_e = wr_ref.shape[0]
    hidden = w2_ref.shape[2]

    @pl.when(e == 0)
    def _():
        zt = jnp.dot(wr_ref[...], xt32_ref[...],
                     preferred_element_type=jnp.float32)
        iota = jax.lax.broadcasted_iota(jnp.int32, zt.shape, 0)
        m1 = jnp.max(zt, axis=0, keepdims=True)
        i1 = jnp.min(jnp.where(zt == m1, iota, num_e), axis=0, keepdims=True)
        is1 = iota == i1
        z2 = jnp.where(is1, -jnp.inf, zt)
        m2 = jnp.max(z2, axis=0, keepdims=True)
        i2 = jnp.min(jnp.where(z2 == m2, iota, num_e), axis=0, keepdims=True)
        is2 = iota == i2
        t = jnp.exp(m2 - m1)
        g1 = 1.0 / (1.0 + t)
        g2 = t / (1.0 + t)
        gates_s[...] = jnp.where(is1, g1, jnp.where(is2, g2, 0.0))

    iota_e = jax.lax.broadcasted_iota(jnp.int32, gates_s.shape, 0)
    w_row = jnp.sum(jnp.where(iota_e == e, gates_s[...], 0.0),
                    axis=0, keepdims=True)

    w1bf = w1_ref[0].astype(jnp.bfloat16)
    h = jnp.dot(w1bf, xtbf_ref[...], preferred_element_type=jnp.float32)
    a = h[:hidden]
    b = h[hidden:]
    act = (a * jax.nn.sigmoid(a) * b * w_row).astype(jnp.bfloat16)
    w2bf = w2_ref[0].astype(jnp.bfloat16)
    y = jnp.dot(w2bf, act, preferred_element_type=jnp.float32)

    @pl.when(e == 0)
    def _():
        o_ref[...] = y

    @pl.when(e > 0)
    def _():
        o_ref[...] += y


def kernel(x, Wr, W1, W2):
    B, T, D = x.shape
    num_e, two_h, _ = W1.shape
    hidden = W2.shape[2]
    xt = x.reshape(T, D).T
    xtbf = xt.astype(jnp.bfloat16)

    out_t = pl.pallas_call(
        _moe_body,
        grid=(num_e,),
        in_specs=[
            pl.BlockSpec((D, T), lambda e: (0, 0)),
            pl.BlockSpec((D, T), lambda e: (0, 0)),
            pl.BlockSpec((num_e, D), lambda e: (0, 0)),
            pl.BlockSpec((1, two_h, D), lambda e: (e, 0, 0)),
            pl.BlockSpec((1, D, hidden), lambda e: (e, 0, 0)),
        ],
        out_specs=pl.BlockSpec((D, T), lambda e: (0, 0)),
        out_shape=jax.ShapeDtypeStruct((D, T), jnp.float32),
        scratch_shapes=[pltpu.VMEM((num_e, T), jnp.float32)],
    )(xt, xtbf, Wr, W1, W2)

    return out_t.T.reshape(B, T, D)


# speedup vs baseline: 2.3247x; 1.4841x over previous
import jax
import jax.numpy as jnp
from jax.experimental import pallas as pl
from jax.experimental.pallas import tpu as pltpu

_DN_RT = (((1,), (1,)), ((), ()))


def _moe_body(x_ref, wr_ref, w1_ref, w2_ref, o_ref, gates_s, xbf_s):
    e = pl.program_id(0)
    num_e = wr_ref.shape[0]
    hidden = w2_ref.shape[2]

    @pl.when(e == 0)
    def _():
        xbf_s[...] = x_ref[...].astype(jnp.bfloat16)
        z = jax.lax.dot_general(x_ref[...], wr_ref[...], _DN_RT,
                                preferred_element_type=jnp.float32)
        iota = jax.lax.broadcasted_iota(jnp.int32, z.shape, 1)
        m1 = jnp.max(z, axis=1, keepdims=True)
        i1 = jnp.min(jnp.where(z == m1, iota, num_e), axis=1, keepdims=True)
        is1 = iota == i1
        z2 = jnp.where(is1, -jnp.inf, z)
        m2 = jnp.max(z2, axis=1, keepdims=True)
        i2 = jnp.min(jnp.where(z2 == m2, iota, num_e), axis=1, keepdims=True)
        is2 = iota == i2
        t = jnp.exp(m2 - m1)
        g1 = 1.0 / (1.0 + t)
        g2 = t / (1.0 + t)
        gates_s[...] = jnp.where(is1, g1, jnp.where(is2, g2, 0.0))

    iota_e = jax.lax.broadcasted_iota(jnp.int32, gates_s.shape, 1)
    w_col = jnp.sum(jnp.where(iota_e == e, gates_s[...], 0.0),
                    axis=1, keepdims=True)

    w1bf = w1_ref[0].astype(jnp.bfloat16)
    h = jax.lax.dot_general(xbf_s[...], w1bf, _DN_RT,
                            preferred_element_type=jnp.float32)
    a = h[:, :hidden]
    b = h[:, hidden:]
    act = (a * jax.nn.sigmoid(a) * b * w_col).astype(jnp.bfloat16)
    w2bf = w2_ref[0].astype(jnp.bfloat16)
    y = jax.lax.dot_general(act, w2bf, _DN_RT,
                            preferred_element_type=jnp.float32)

    @pl.when(e == 0)
    def _():
        o_ref[...] = y

    @pl.when(e > 0)
    def _():
        o_ref[...] += y


def kernel(x, Wr, W1, W2):
    B, T, D = x.shape
    num_e, two_h, _ = W1.shape
    hidden = W2.shape[2]
    x2 = x.reshape(T, D)

    out = pl.pallas_call(
        _moe_body,
        grid=(num_e,),
        in_specs=[
            pl.BlockSpec((T, D), lambda e: (0, 0)),
            pl.BlockSpec((num_e, D), lambda e: (0, 0)),
            pl.BlockSpec((1, two_h, D), lambda e: (e, 0, 0)),
            pl.BlockSpec((1, D, hidden), lambda e: (e, 0, 0)),
        ],
        out_specs=pl.BlockSpec((T, D), lambda e: (0, 0)),
        out_shape=jax.ShapeDtypeStruct((T, D), jnp.float32),
        scratch_shapes=[
            pltpu.VMEM((T, num_e), jnp.float32),
            pltpu.VMEM((T, D), jnp.bfloat16),
        ],
    )(x2, Wr, W1, W2)

    return out.reshape(B, T, D)
